# baseline (speedup 1.0000x reference)
import functools

import jax
import jax.numpy as jnp
from jax import lax
from jax.experimental import pallas as pl

N_TOKENS = 32768
EMB = 768
NE = 8
ROWS = 1024


def _router_block(x_ref, w_ref, logits_ref, probs_ref, idx_ref, wts_ref):
    x = x_ref[...]
    w = w_ref[...]
    lg = lax.dot_general(x, w, (((1,), (0,)), ((), ())),
                         preferred_element_type=jnp.float32)
    logits_ref[...] = lg
    m = jnp.max(lg, axis=1, keepdims=True)
    e = jnp.exp(lg - m)
    p = e / jnp.sum(e, axis=1, keepdims=True)
    probs_ref[...] = p

    iota = lax.broadcasted_iota(jnp.int32, (ROWS, NE), 1)
    p1 = jnp.max(p, axis=1, keepdims=True)
    i1 = jnp.min(jnp.where(p == p1, iota, NE), axis=1, keepdims=True)
    masked = jnp.where(iota == i1, -jnp.inf, p)
    p2 = jnp.max(masked, axis=1, keepdims=True)
    i2 = jnp.min(jnp.where(masked == p2, iota, NE), axis=1, keepdims=True)
    denom = jnp.maximum(p1 + p2, 1e-8)
    idx_ref[...] = jnp.concatenate([i1, i2], axis=1)
    wts_ref[...] = jnp.concatenate([p1 / denom, p2 / denom], axis=1)


@jax.jit
def kernel(noise_clock_emb, route_weight):
    grid = N_TOKENS // ROWS
    out_shape = (
        jax.ShapeDtypeStruct((N_TOKENS, NE), jnp.float32),
        jax.ShapeDtypeStruct((N_TOKENS, NE), jnp.float32),
        jax.ShapeDtypeStruct((N_TOKENS, 2), jnp.int32),
        jax.ShapeDtypeStruct((N_TOKENS, 2), jnp.float32),
    )
    logits, probs, idx, wts = pl.pallas_call(
        _router_block,
        grid=(grid,),
        in_specs=[
            pl.BlockSpec((ROWS, EMB), lambda i: (i, 0)),
            pl.BlockSpec((EMB, NE), lambda i: (0, 0)),
        ],
        out_specs=[
            pl.BlockSpec((ROWS, NE), lambda i: (i, 0)),
            pl.BlockSpec((ROWS, NE), lambda i: (i, 0)),
            pl.BlockSpec((ROWS, 2), lambda i: (i, 0)),
            pl.BlockSpec((ROWS, 2), lambda i: (i, 0)),
        ],
        out_shape=out_shape,
    )(noise_clock_emb, route_weight)
    return (logits, probs, idx, wts)


# speedup vs baseline: 1.0354x; 1.0354x over previous
import functools

import jax
import jax.numpy as jnp
from jax import lax
from jax.experimental import pallas as pl

N_TOKENS = 32768
EMB = 768
NE = 8
ROWS = 1024


def _router_block(x_ref, w_ref, logits_ref, probs_ref, idx_ref, wts_ref):
    x = x_ref[...]
    w = w_ref[...]
    lg = lax.dot_general(x, w, (((1,), (0,)), ((), ())),
                         preferred_element_type=jnp.float32)
    logits_ref[...] = lg

    lgT = lax.dot_general(w, x, (((0,), (1,)), ((), ())),
                          preferred_element_type=jnp.float32)
    mT = jnp.max(lgT, axis=0, keepdims=True)
    eT = jnp.exp(lgT - mT)
    pT = eT / jnp.sum(eT, axis=0, keepdims=True)
    probs_ref[...] = pT.T

    iota = lax.broadcasted_iota(jnp.int32, (NE, ROWS), 0)
    p1 = jnp.max(pT, axis=0, keepdims=True)
    i1 = jnp.min(jnp.where(pT == p1, iota, NE), axis=0, keepdims=True)
    masked = jnp.where(iota == i1, -jnp.inf, pT)
    p2 = jnp.max(masked, axis=0, keepdims=True)
    i2 = jnp.min(jnp.where(masked == p2, iota, NE), axis=0, keepdims=True)
    denom = jnp.maximum(p1 + p2, 1e-8)
    idx_ref[...] = jnp.concatenate([i1, i2], axis=0).T
    wts_ref[...] = jnp.concatenate([p1 / denom, p2 / denom], axis=0).T


@jax.jit
def kernel(noise_clock_emb, route_weight):
    grid = N_TOKENS // ROWS
    out_shape = (
        jax.ShapeDtypeStruct((N_TOKENS, NE), jnp.float32),
        jax.ShapeDtypeStruct((N_TOKENS, NE), jnp.float32),
        jax.ShapeDtypeStruct((N_TOKENS, 2), jnp.int32),
        jax.ShapeDtypeStruct((N_TOKENS, 2), jnp.float32),
    )
    logits, probs, idx, wts = pl.pallas_call(
        _router_block,
        grid=(grid,),
        in_specs=[
            pl.BlockSpec((ROWS, EMB), lambda i: (i, 0)),
            pl.BlockSpec((EMB, NE), lambda i: (0, 0)),
        ],
        out_specs=[
            pl.BlockSpec((ROWS, NE), lambda i: (i, 0)),
            pl.BlockSpec((ROWS, NE), lambda i: (i, 0)),
            pl.BlockSpec((ROWS, 2), lambda i: (i, 0)),
            pl.BlockSpec((ROWS, 2), lambda i: (i, 0)),
        ],
        out_shape=out_shape,
    )(noise_clock_emb, route_weight)
    return (logits, probs, idx, wts)


# speedup vs baseline: 1.1563x; 1.1168x over previous
import functools

import jax
import jax.numpy as jnp
from jax import lax
from jax.experimental import pallas as pl

N_TOKENS = 32768
EMB = 768
NE = 8
ROWS = 4096


def _router_block(x_ref, w_ref, logits_ref, probs_ref, idx_ref, wts_ref):
    x = x_ref[...]
    w = w_ref[...]
    lg = lax.dot_general(x, w, (((1,), (0,)), ((), ())),
                         preferred_element_type=jnp.float32)
    logits_ref[...] = lg

    lgT = lax.dot_general(w, x, (((0,), (1,)), ((), ())),
                          preferred_element_type=jnp.float32)
    mT = jnp.max(lgT, axis=0, keepdims=True)
    eT = jnp.exp(lgT - mT)
    pT = eT / jnp.sum(eT, axis=0, keepdims=True)
    probs_ref[...] = pT.T

    iota = lax.broadcasted_iota(jnp.int32, (NE, ROWS), 0)
    p1 = jnp.max(pT, axis=0, keepdims=True)
    i1 = jnp.min(jnp.where(pT == p1, iota, NE), axis=0, keepdims=True)
    masked = jnp.where(iota == i1, -jnp.inf, pT)
    p2 = jnp.max(masked, axis=0, keepdims=True)
    i2 = jnp.min(jnp.where(masked == p2, iota, NE), axis=0, keepdims=True)
    denom = jnp.maximum(p1 + p2, 1e-8)
    idx_ref[...] = jnp.concatenate([i1, i2], axis=0).T
    wts_ref[...] = jnp.concatenate([p1 / denom, p2 / denom], axis=0).T


@jax.jit
def kernel(noise_clock_emb, route_weight):
    grid = N_TOKENS // ROWS
    out_shape = (
        jax.ShapeDtypeStruct((N_TOKENS, NE), jnp.float32),
        jax.ShapeDtypeStruct((N_TOKENS, NE), jnp.float32),
        jax.ShapeDtypeStruct((N_TOKENS, 2), jnp.int32),
        jax.ShapeDtypeStruct((N_TOKENS, 2), jnp.float32),
    )
    logits, probs, idx, wts = pl.pallas_call(
        _router_block,
        grid=(grid,),
        in_specs=[
            pl.BlockSpec((ROWS, EMB), lambda i: (i, 0)),
            pl.BlockSpec((EMB, NE), lambda i: (0, 0)),
        ],
        out_specs=[
            pl.BlockSpec((ROWS, NE), lambda i: (i, 0)),
            pl.BlockSpec((ROWS, NE), lambda i: (i, 0)),
            pl.BlockSpec((ROWS, 2), lambda i: (i, 0)),
            pl.BlockSpec((ROWS, 2), lambda i: (i, 0)),
        ],
        out_shape=out_shape,
    )(noise_clock_emb, route_weight)
    return (logits, probs, idx, wts)


# speedup vs baseline: 1.1631x; 1.0059x over previous
import functools

import jax
import jax.numpy as jnp
from jax import lax
from jax.experimental import pallas as pl

N_TOKENS = 32768
EMB = 768
NE = 8
NSPLIT = 4
Q = 1024
STEP = NSPLIT * Q


def _router_quarter(x, w, j, logits_ref, probs_ref, idx_ref, wts_ref):
    lg = lax.dot_general(x, w, (((1,), (0,)), ((), ())),
                         preferred_element_type=jnp.float32)
    logits_ref[pl.ds(j * Q, Q), :] = lg

    lgT = lax.dot_general(w, x, (((0,), (1,)), ((), ())),
                          preferred_element_type=jnp.float32)
    mT = jnp.max(lgT, axis=0, keepdims=True)
    eT = jnp.exp(lgT - mT)
    pT = eT / jnp.sum(eT, axis=0, keepdims=True)
    probs_ref[pl.ds(j * Q, Q), :] = pT.T

    iota = lax.broadcasted_iota(jnp.int32, (NE, Q), 0)
    p1 = jnp.max(pT, axis=0, keepdims=True)
    i1 = jnp.min(jnp.where(pT == p1, iota, NE), axis=0, keepdims=True)
    masked = jnp.where(iota == i1, -jnp.inf, pT)
    p2 = jnp.max(masked, axis=0, keepdims=True)
    i2 = jnp.min(jnp.where(masked == p2, iota, NE), axis=0, keepdims=True)
    denom = jnp.maximum(p1 + p2, 1e-8)
    idx_ref[pl.ds(j * Q, Q), :] = jnp.concatenate([i1, i2], axis=0).T
    wts_ref[pl.ds(j * Q, Q), :] = jnp.concatenate(
        [p1 / denom, p2 / denom], axis=0).T


def _router_block(*refs):
    x_refs = refs[:NSPLIT]
    w_ref = refs[NSPLIT]
    logits_ref, probs_ref, idx_ref, wts_ref = refs[NSPLIT + 1:]
    w = w_ref[...]
    for j in range(NSPLIT):
        _router_quarter(x_refs[j][...], w, j,
                        logits_ref, probs_ref, idx_ref, wts_ref)


@jax.jit
def kernel(noise_clock_emb, route_weight):
    grid = N_TOKENS // STEP

    def band(j):
        return pl.BlockSpec((Q, EMB), lambda i, j=j: (NSPLIT * i + j, 0))

    in_specs = [band(j) for j in range(NSPLIT)]
    in_specs.append(pl.BlockSpec((EMB, NE), lambda i: (0, 0)))
    out_specs = [
        pl.BlockSpec((STEP, NE), lambda i: (i, 0)),
        pl.BlockSpec((STEP, NE), lambda i: (i, 0)),
        pl.BlockSpec((STEP, 2), lambda i: (i, 0)),
        pl.BlockSpec((STEP, 2), lambda i: (i, 0)),
    ]
    out_shape = (
        jax.ShapeDtypeStruct((N_TOKENS, NE), jnp.float32),
        jax.ShapeDtypeStruct((N_TOKENS, NE), jnp.float32),
        jax.ShapeDtypeStruct((N_TOKENS, 2), jnp.int32),
        jax.ShapeDtypeStruct((N_TOKENS, 2), jnp.float32),
    )
    logits, probs, idx, wts = pl.pallas_call(
        _router_block,
        grid=(grid,),
        in_specs=in_specs,
        out_specs=out_specs,
        out_shape=out_shape,
    )(*([noise_clock_emb] * NSPLIT), route_weight)
    return (logits, probs, idx, wts)


# speedup vs baseline: 3.0471x; 2.6198x over previous
import functools

import jax
import jax.numpy as jnp
from jax import lax
from jax.experimental import pallas as pl

N_TOKENS = 32768
EMB = 768
NE = 8
NSPLIT = 4
Q = 1024
STEP = NSPLIT * Q


def _router_quarter(x, w, j, logitsT_ref, probsT_ref, idxT_ref, wtsT_ref):
    lgT = lax.dot_general(w, x, (((0,), (1,)), ((), ())),
                          preferred_element_type=jnp.float32)
    logitsT_ref[:, pl.ds(j * Q, Q)] = lgT
    mT = jnp.max(lgT, axis=0, keepdims=True)
    eT = jnp.exp(lgT - mT)
    pT = eT / jnp.sum(eT, axis=0, keepdims=True)
    probsT_ref[:, pl.ds(j * Q, Q)] = pT

    iota = lax.broadcasted_iota(jnp.int32, (NE, Q), 0)
    p1 = jnp.max(pT, axis=0, keepdims=True)
    i1 = jnp.min(jnp.where(pT == p1, iota, NE), axis=0, keepdims=True)
    masked = jnp.where(iota == i1, -jnp.inf, pT)
    p2 = jnp.max(masked, axis=0, keepdims=True)
    i2 = jnp.min(jnp.where(masked == p2, iota, NE), axis=0, keepdims=True)
    denom = jnp.maximum(p1 + p2, 1e-8)
    idxT_ref[:, pl.ds(j * Q, Q)] = jnp.concatenate([i1, i2], axis=0)
    wtsT_ref[:, pl.ds(j * Q, Q)] = jnp.concatenate(
        [p1 / denom, p2 / denom], axis=0)


def _router_block(*refs):
    x_refs = refs[:NSPLIT]
    w_ref = refs[NSPLIT]
    logitsT_ref, probsT_ref, idxT_ref, wtsT_ref = refs[NSPLIT + 1:]
    w = w_ref[...]
    for j in range(NSPLIT):
        _router_quarter(x_refs[j][...], w, j,
                        logitsT_ref, probsT_ref, idxT_ref, wtsT_ref)


@jax.jit
def kernel(noise_clock_emb, route_weight):
    grid = N_TOKENS // STEP

    def band(j):
        return pl.BlockSpec((Q, EMB), lambda i, j=j: (NSPLIT * i + j, 0))

    in_specs = [band(j) for j in range(NSPLIT)]
    in_specs.append(pl.BlockSpec((EMB, NE), lambda i: (0, 0)))
    out_specs = [
        pl.BlockSpec((NE, STEP), lambda i: (0, i)),
        pl.BlockSpec((NE, STEP), lambda i: (0, i)),
        pl.BlockSpec((2, STEP), lambda i: (0, i)),
        pl.BlockSpec((2, STEP), lambda i: (0, i)),
    ]
    out_shape = (
        jax.ShapeDtypeStruct((NE, N_TOKENS), jnp.float32),
        jax.ShapeDtypeStruct((NE, N_TOKENS), jnp.float32),
        jax.ShapeDtypeStruct((2, N_TOKENS), jnp.int32),
        jax.ShapeDtypeStruct((2, N_TOKENS), jnp.float32),
    )
    logitsT, probsT, idxT, wtsT = pl.pallas_call(
        _router_block,
        grid=(grid,),
        in_specs=in_specs,
        out_specs=out_specs,
        out_shape=out_shape,
    )(*([noise_clock_emb] * NSPLIT), route_weight)
    return (logitsT.T, probsT.T, idxT.T, wtsT.T)


# speedup vs baseline: 3.0715x; 1.0080x over previous
import functools

import jax
import jax.numpy as jnp
from jax import lax
from jax.experimental import pallas as pl

N_TOKENS = 32768
EMB = 768
NE = 8
NSPLIT = 1
Q = 4096
STEP = NSPLIT * Q


def _router_quarter(x, w, j, logitsT_ref, probsT_ref, idxT_ref, wtsT_ref):
    lgT = lax.dot_general(w, x, (((0,), (1,)), ((), ())),
                          preferred_element_type=jnp.float32)
    logitsT_ref[:, pl.ds(j * Q, Q)] = lgT
    mT = jnp.max(lgT, axis=0, keepdims=True)
    eT = jnp.exp(lgT - mT)
    pT = eT / jnp.sum(eT, axis=0, keepdims=True)
    probsT_ref[:, pl.ds(j * Q, Q)] = pT

    iota = lax.broadcasted_iota(jnp.int32, (NE, Q), 0)
    p1 = jnp.max(pT, axis=0, keepdims=True)
    i1 = jnp.min(jnp.where(pT == p1, iota, NE), axis=0, keepdims=True)
    masked = jnp.where(iota == i1, -jnp.inf, pT)
    p2 = jnp.max(masked, axis=0, keepdims=True)
    i2 = jnp.min(jnp.where(masked == p2, iota, NE), axis=0, keepdims=True)
    denom = jnp.maximum(p1 + p2, 1e-8)
    idxT_ref[:, pl.ds(j * Q, Q)] = jnp.concatenate([i1, i2], axis=0)
    wtsT_ref[:, pl.ds(j * Q, Q)] = jnp.concatenate(
        [p1 / denom, p2 / denom], axis=0)


def _router_block(*refs):
    x_refs = refs[:NSPLIT]
    w_ref = refs[NSPLIT]
    logitsT_ref, probsT_ref, idxT_ref, wtsT_ref = refs[NSPLIT + 1:]
    w = w_ref[...]
    for j in range(NSPLIT):
        _router_quarter(x_refs[j][...], w, j,
                        logitsT_ref, probsT_ref, idxT_ref, wtsT_ref)


@jax.jit
def kernel(noise_clock_emb, route_weight):
    grid = N_TOKENS // STEP

    def band(j):
        return pl.BlockSpec((Q, EMB), lambda i, j=j: (NSPLIT * i + j, 0))

    in_specs = [band(j) for j in range(NSPLIT)]
    in_specs.append(pl.BlockSpec((EMB, NE), lambda i: (0, 0)))
    out_specs = [
        pl.BlockSpec((NE, STEP), lambda i: (0, i)),
        pl.BlockSpec((NE, STEP), lambda i: (0, i)),
        pl.BlockSpec((2, STEP), lambda i: (0, i)),
        pl.BlockSpec((2, STEP), lambda i: (0, i)),
    ]
    out_shape = (
        jax.ShapeDtypeStruct((NE, N_TOKENS), jnp.float32),
        jax.ShapeDtypeStruct((NE, N_TOKENS), jnp.float32),
        jax.ShapeDtypeStruct((2, N_TOKENS), jnp.int32),
        jax.ShapeDtypeStruct((2, N_TOKENS), jnp.float32),
    )
    logitsT, probsT, idxT, wtsT = pl.pallas_call(
        _router_block,
        grid=(grid,),
        in_specs=in_specs,
        out_specs=out_specs,
        out_shape=out_shape,
    )(*([noise_clock_emb] * NSPLIT), route_weight)
    return (logitsT.T, probsT.T, idxT.T, wtsT.T)
